# baseline (speedup 1.0000x reference)
import functools

import jax
import jax.numpy as jnp
from jax import lax
from jax.experimental import pallas as pl
from jax.experimental.pallas import tpu as pltpu
from jax.experimental.pallas import tpu_sc as plsc

N = 10000
E = 320000
D = 128
NP = 10240
NC = 2
NS = 16
NW = NC * NS
CK = 32
EW_T = 10112
DCH = EW_T // CK
EP = NW * EW_T
CH0 = 432
CH1 = 200
CHMX = max(CH0, CH1)
OFF1 = NS * CH0 * CK
RB = NP // NS

_mesh = plsc.VectorSubcoreMesh(core_axis_name="c", subcore_axis_name="s")


def _rsqrt_newton(v):
    r = jnp.full((16,), 1.0, jnp.float32)
    for _ in range(16):
        m = v > 4.0
        v = jnp.where(m, v * 0.25, v)
        r = jnp.where(m, r * 0.5, r)
    y = 1.0 / v
    for _ in range(6):
        y = y * (1.5 - 0.5 * v * y * y)
    return y * r


@functools.partial(
    pl.kernel,
    out_type=jax.ShapeDtypeStruct((NP,), jnp.float32),
    mesh=_mesh,
    scratch_types=[
        pltpu.VMEM((EW_T,), jnp.int32),
        pltpu.VMEM((EW_T,), jnp.float32),
        pltpu.VMEM((1, CK), jnp.int32),
        pltpu.VMEM((RB,), jnp.float32),
        pltpu.VMEM_SHARED((NP,), jnp.float32),
    ],
)
def _deg_dis(rcp, ewp, dis_out, rc_v, ew_v, colrow, dbuf, sdegf):
    cid = lax.axis_index("c")
    sid = lax.axis_index("s")

    for t in range(RB // 16):
        dbuf[pl.ds(t * 16, 16)] = jnp.zeros((16,), jnp.float32)
    pltpu.sync_copy(dbuf, sdegf.at[pl.ds(sid * RB, RB)])
    plsc.subcore_barrier()

    def do_shard(w):
        pltpu.sync_copy(rcp.at[pl.ds(w * EW_T, EW_T)], rc_v)
        pltpu.sync_copy(ewp.at[pl.ds(w * EW_T, EW_T)], ew_v)

        def chunk(j, _):
            for g in range(CK // 16):
                p16 = rc_v[pl.ds(j * CK + g * 16, 16)]
                colrow[0, pl.ds(g * 16, 16)] = lax.shift_right_logical(p16, 14)
            pltpu.sync_copy(ew_v.at[pl.ds(j * CK, CK)],
                            sdegf.at[colrow.at[0]], add=True)
            return 0

        lax.fori_loop(0, DCH, chunk, 0)

    do_shard(sid * 2)
    do_shard(sid * 2 + 1)
    plsc.subcore_barrier()

    pltpu.sync_copy(sdegf.at[pl.ds(sid * RB, RB)], dbuf)
    for t in range(RB // 16):
        v = dbuf[pl.ds(t * 16, 16)] + 1.0
        dbuf[pl.ds(t * 16, 16)] = _rsqrt_newton(v)

    @pl.when(cid == 0)
    def _():
        pltpu.sync_copy(dbuf, dis_out.at[pl.ds(sid * RB, RB)])


@functools.partial(
    pl.kernel,
    out_type=jax.ShapeDtypeStruct((NC, NP, 128), jnp.float32),
    mesh=_mesh,
    scratch_types=[
        pltpu.VMEM((CHMX * CK,), jnp.int32),
        pltpu.VMEM((CHMX * CK + 16,), jnp.float32),
        pltpu.VMEM((2, CK), jnp.int32),
        pltpu.VMEM((2, CK), jnp.int32),
        pltpu.VMEM((2, CK, 128), jnp.float32),
        pltpu.VMEM_SHARED((NP, 128), jnp.float32),
        pltpu.SemaphoreType.DMA,
        pltpu.SemaphoreType.DMA,
    ],
)
def _propagate(hs, rcp, ewp, out,
               rc_v, ew_v, rowrow, colrow, gbuf, acc, gsem, ssem):
    cid = lax.axis_index("c")
    sid = lax.axis_index("s")
    nch = jnp.where(cid == 0, CH0, CH1)

    def zrow(i, _):
        for g in range(8):
            gbuf[0, i, pl.ds(g * 16, 16)] = jnp.zeros((16,), jnp.float32)
        return 0

    lax.fori_loop(0, CK, zrow, 0)
    for k in range(RB // CK):
        pltpu.sync_copy(gbuf.at[0], acc.at[pl.ds(sid * RB + k * CK, CK)])

    @pl.when(cid == 0)
    def _():
        pltpu.sync_copy(rcp.at[pl.ds(sid * CH0 * CK, CH0 * CK)],
                        rc_v.at[pl.ds(0, CH0 * CK)])
        pltpu.sync_copy(ewp.at[pl.ds(sid * CH0 * CK, CH0 * CK)],
                        ew_v.at[pl.ds(0, CH0 * CK)])

    @pl.when(cid == 1)
    def _():
        pltpu.sync_copy(rcp.at[pl.ds(OFF1 + sid * CH1 * CK, CH1 * CK)],
                        rc_v.at[pl.ds(0, CH1 * CK)])
        pltpu.sync_copy(ewp.at[pl.ds(OFF1 + sid * CH1 * CK, CH1 * CK)],
                        ew_v.at[pl.ds(0, CH1 * CK)])

    plsc.subcore_barrier()

    def fill_rowrow(j):
        b = jnp.remainder(j, 2)
        for g in range(CK // 16):
            p16 = rc_v[pl.ds(j * CK + g * 16, 16)]
            rowrow[b, pl.ds(g * 16, 16)] = jnp.bitwise_and(p16, 16383)

    def issue(j):
        b = jnp.remainder(j, 2)
        pltpu.async_copy(hs.at[rowrow.at[b]], gbuf.at[b], gsem)

    fill_rowrow(0)
    issue(0)

    def chunk(j, _):
        b = jnp.remainder(j, 2)
        bo = 1 - b
        pltpu.make_async_copy(hs.at[rowrow.at[b]], gbuf.at[b], gsem).wait()

        @pl.when(j >= 1)
        def _():
            pltpu.make_async_copy(gbuf.at[bo], acc.at[colrow.at[bo]],
                                  ssem).wait()

        @pl.when(j + 1 < nch)
        def _():
            fill_rowrow(j + 1)
            issue(j + 1)

        def scale(e, _):
            s = ew_v[pl.ds(j * CK + e, 16)][0]
            for g in range(8):
                sl = pl.ds(g * 16, 16)
                gbuf[b, e, sl] = gbuf[b, e, sl] * s
            return 0

        lax.fori_loop(0, CK, scale, 0, unroll=2)

        for g in range(CK // 16):
            p16 = rc_v[pl.ds(j * CK + g * 16, 16)]
            colrow[b, pl.ds(g * 16, 16)] = lax.shift_right_logical(p16, 14)
        pltpu.async_copy(gbuf.at[b], acc.at[colrow.at[b]], ssem, add=True)
        return 0

    lax.fori_loop(0, nch, chunk, 0)
    bl = jnp.remainder(nch - 1, 2)
    pltpu.make_async_copy(gbuf.at[bl], acc.at[colrow.at[bl]], ssem).wait()
    plsc.subcore_barrier()

    pltpu.sync_copy(acc.at[pl.ds(sid * RB, RB)], out.at[cid, pl.ds(sid * RB, RB)])


_BLK = 1024


def _mm_scale_body(x_ref, w_ref, dis_ref, o_ref):
    o_ref[...] = (jnp.dot(x_ref[...], w_ref[...],
                          preferred_element_type=jnp.float32) * dis_ref[...])


def _mid_body(p0_ref, p1_ref, hs_ref, dis_ref, b_ref, w_ref, o_ref):
    z = (p0_ref[...] + p1_ref[...] + hs_ref[...]) * dis_ref[...] + b_ref[...]
    r = jnp.maximum(z, 0.0)
    o_ref[...] = (jnp.dot(r, w_ref[...],
                          preferred_element_type=jnp.float32) * dis_ref[...])


def _final_body(q0_ref, q1_ref, hs_ref, dis_ref, b_ref, o_ref):
    o_ref[...] = ((q0_ref[...] + q1_ref[...] + hs_ref[...]) * dis_ref[...]
                  + b_ref[...])


_ROWS = pl.BlockSpec((_BLK, D), lambda i: (i, 0))
_HSROWS = pl.BlockSpec((_BLK, D), lambda i: (i, 0))
_DISB = pl.BlockSpec((_BLK, 1), lambda i: (i, 0))
_WMAT = pl.BlockSpec((D, D), lambda i: (0, 0))
_BIAS = pl.BlockSpec((1, D), lambda i: (0, 0))


def _tc_mm_scale(x, W, dis2d):
    return pl.pallas_call(
        _mm_scale_body,
        grid=(NP // _BLK,),
        in_specs=[_ROWS, _WMAT, _DISB],
        out_specs=_ROWS,
        out_shape=jax.ShapeDtypeStruct((NP, D), jnp.float32),
    )(x, W, dis2d)


def _tc_mid(p0, p1, hs, dis2d, b, W):
    return pl.pallas_call(
        _mid_body,
        grid=(NP // _BLK,),
        in_specs=[_ROWS, _ROWS, _ROWS, _DISB, _BIAS, _WMAT],
        out_specs=_ROWS,
        out_shape=jax.ShapeDtypeStruct((NP, D), jnp.float32),
    )(p0, p1, hs, dis2d, b, W)


def _tc_final(q0, q1, hs, dis2d, b):
    return pl.pallas_call(
        _final_body,
        grid=(NP // _BLK,),
        in_specs=[_ROWS, _ROWS, _ROWS, _DISB, _BIAS],
        out_specs=_ROWS,
        out_shape=jax.ShapeDtypeStruct((NP, D), jnp.float32),
    )(q0, q1, hs, dis2d, b)


def kernel(x, edge_index, edge_weight, W1, b1, W2, b2):
    row = edge_index[0].astype(jnp.int32)
    col = edge_index[1].astype(jnp.int32)
    ew = edge_weight.astype(jnp.float32)
    pad = EP - E
    rc = jnp.bitwise_or(row, lax.shift_left(col, 14))
    rcp = jnp.concatenate([rc, jnp.zeros((pad,), jnp.int32)])
    ewp = jnp.concatenate([ew, jnp.zeros((pad,), jnp.float32)])
    xp = jnp.concatenate([x.astype(jnp.float32),
                          jnp.zeros((NP - N, D), jnp.float32)], axis=0)

    dis = _deg_dis(rcp, ewp)
    dis2d = dis.reshape(NP, 1)
    hs1 = _tc_mm_scale(xp, W1.astype(jnp.float32), dis2d)
    P = _propagate(hs1, rcp, ewp)
    hs2 = _tc_mid(P[0], P[1], hs1, dis2d, b1.reshape(1, D).astype(jnp.float32),
                  W2.astype(jnp.float32))
    Q = _propagate(hs2, rcp, ewp)
    out = _tc_final(Q[0], Q[1], hs2, dis2d, b2.reshape(1, D).astype(jnp.float32))
    return out[:N]


# speedup vs baseline: 1.4040x; 1.4040x over previous
import functools

import jax
import jax.numpy as jnp
from jax import lax
from jax.experimental import pallas as pl
from jax.experimental.pallas import tpu as pltpu
from jax.experimental.pallas import tpu_sc as plsc

N = 10000
E = 320000
D = 128
NP = 10240
NC = 2
NS = 16
NW = NC * NS
CK = 128
EW_T = 10112
DCH = EW_T // CK
EP = NW * EW_T
CH0 = 108
CH1 = 50
HB0 = CH0 // 2
HB1 = CH1 // 2
OFF1 = NS * CH0 * CK
RB = NP // NS

_mesh = plsc.VectorSubcoreMesh(core_axis_name="c", subcore_axis_name="s")


def _rsqrt_newton(v):
    r = jnp.full((16,), 1.0, jnp.float32)
    for _ in range(16):
        m = v > 4.0
        v = jnp.where(m, v * 0.25, v)
        r = jnp.where(m, r * 0.5, r)
    y = 1.0 / v
    for _ in range(6):
        y = y * (1.5 - 0.5 * v * y * y)
    return y * r


@functools.partial(
    pl.kernel,
    out_type=jax.ShapeDtypeStruct((NP,), jnp.float32),
    mesh=_mesh,
    scratch_types=[
        pltpu.VMEM((EW_T,), jnp.int32),
        pltpu.VMEM((EW_T,), jnp.float32),
        pltpu.VMEM((1, CK), jnp.int32),
        pltpu.VMEM((RB,), jnp.float32),
        pltpu.VMEM_SHARED((NP,), jnp.float32),
    ],
)
def _deg_dis(rcp, ewp, dis_out, rc_v, ew_v, colrow, dbuf, sdegf):
    cid = lax.axis_index("c")
    sid = lax.axis_index("s")

    for t in range(RB // 16):
        dbuf[pl.ds(t * 16, 16)] = jnp.zeros((16,), jnp.float32)
    pltpu.sync_copy(dbuf, sdegf.at[pl.ds(sid * RB, RB)])
    plsc.subcore_barrier()

    def do_shard(w):
        pltpu.sync_copy(rcp.at[pl.ds(w * EW_T, EW_T)], rc_v)
        pltpu.sync_copy(ewp.at[pl.ds(w * EW_T, EW_T)], ew_v)

        def chunk(j, _):
            for g in range(CK // 16):
                p16 = rc_v[pl.ds(j * CK + g * 16, 16)]
                colrow[0, pl.ds(g * 16, 16)] = lax.shift_right_logical(p16, 14)
            pltpu.sync_copy(ew_v.at[pl.ds(j * CK, CK)],
                            sdegf.at[colrow.at[0]], add=True)
            return 0

        lax.fori_loop(0, DCH, chunk, 0)

    do_shard(sid * 2)
    do_shard(sid * 2 + 1)
    plsc.subcore_barrier()

    pltpu.sync_copy(sdegf.at[pl.ds(sid * RB, RB)], dbuf)
    for t in range(RB // 16):
        v = dbuf[pl.ds(t * 16, 16)] + 1.0
        dbuf[pl.ds(t * 16, 16)] = _rsqrt_newton(v)

    @pl.when(cid == 0)
    def _():
        pltpu.sync_copy(dbuf, dis_out.at[pl.ds(sid * RB, RB)])


@functools.partial(
    pl.kernel,
    out_type=jax.ShapeDtypeStruct((NC, NP, 128), jnp.float32),
    mesh=_mesh,
    scratch_types=[
        pltpu.VMEM((HB0 * CK,), jnp.int32),
        pltpu.VMEM((HB0 * CK + 16,), jnp.float32),
        pltpu.VMEM((2, CK), jnp.int32),
        pltpu.VMEM((2, CK), jnp.int32),
        pltpu.VMEM((2, CK, 128), jnp.float32),
        pltpu.VMEM_SHARED((NP, 128), jnp.float32),
        pltpu.SemaphoreType.DMA,
        pltpu.SemaphoreType.DMA,
    ],
)
def _propagate(hs, rcp, ewp, out,
               rc_v, ew_v, rowrow, colrow, gbuf, acc, gsem, ssem):
    cid = lax.axis_index("c")
    sid = lax.axis_index("s")
    hb = jnp.where(cid == 0, HB0, HB1)

    def zrow(i, _):
        for g in range(8):
            gbuf[0, i, pl.ds(g * 16, 16)] = jnp.zeros((16,), jnp.float32)
        return 0

    lax.fori_loop(0, CK, zrow, 0)
    for k in range(RB // CK):
        pltpu.sync_copy(gbuf.at[0], acc.at[pl.ds(sid * RB + k * CK, CK)])
    plsc.subcore_barrier()

    def fill_rowrow(j):
        b = jnp.remainder(j, 2)
        for g in range(CK // 16):
            p16 = rc_v[pl.ds(j * CK + g * 16, 16)]
            rowrow[b, pl.ds(g * 16, 16)] = jnp.bitwise_and(p16, 16383)

    def issue(j):
        b = jnp.remainder(j, 2)
        pltpu.async_copy(hs.at[rowrow.at[b]], gbuf.at[b], gsem)

    def chunk(j, _):
        b = jnp.remainder(j, 2)
        bo = 1 - b
        pltpu.make_async_copy(hs.at[rowrow.at[b]], gbuf.at[b], gsem).wait()

        @pl.when(j >= 1)
        def _():
            pltpu.make_async_copy(gbuf.at[bo], acc.at[colrow.at[bo]],
                                  ssem).wait()

        @pl.when(j + 1 < hb)
        def _():
            fill_rowrow(j + 1)
            issue(j + 1)

        def scale(e, _):
            s = ew_v[pl.ds(j * CK + e, 16)][0]
            for g in range(8):
                sl = pl.ds(g * 16, 16)
                gbuf[b, e, sl] = gbuf[b, e, sl] * s
            return 0

        lax.fori_loop(0, CK, scale, 0, unroll=2)

        for g in range(CK // 16):
            p16 = rc_v[pl.ds(j * CK + g * 16, 16)]
            colrow[b, pl.ds(g * 16, 16)] = lax.shift_right_logical(p16, 14)
        pltpu.async_copy(gbuf.at[b], acc.at[colrow.at[b]], ssem, add=True)
        return 0

    for p in range(2):
        @pl.when(cid == 0)
        def _():
            off = (sid * CH0 + p * HB0) * CK
            pltpu.sync_copy(rcp.at[pl.ds(off, HB0 * CK)],
                            rc_v.at[pl.ds(0, HB0 * CK)])
            pltpu.sync_copy(ewp.at[pl.ds(off, HB0 * CK)],
                            ew_v.at[pl.ds(0, HB0 * CK)])

        @pl.when(cid == 1)
        def _():
            off = OFF1 + (sid * CH1 + p * HB1) * CK
            pltpu.sync_copy(rcp.at[pl.ds(off, HB1 * CK)],
                            rc_v.at[pl.ds(0, HB1 * CK)])
            pltpu.sync_copy(ewp.at[pl.ds(off, HB1 * CK)],
                            ew_v.at[pl.ds(0, HB1 * CK)])

        fill_rowrow(0)
        issue(0)
        lax.fori_loop(0, hb, chunk, 0)
        bl = jnp.remainder(hb - 1, 2)
        pltpu.make_async_copy(gbuf.at[bl], acc.at[colrow.at[bl]], ssem).wait()

    plsc.subcore_barrier()

    pltpu.sync_copy(acc.at[pl.ds(sid * RB, RB)], out.at[cid, pl.ds(sid * RB, RB)])


_BLK = 1024


def _mm_scale_body(x_ref, w_ref, dis_ref, o_ref):
    o_ref[...] = (jnp.dot(x_ref[...], w_ref[...],
                          preferred_element_type=jnp.float32) * dis_ref[...])


def _mid_body(p0_ref, p1_ref, hs_ref, dis_ref, b_ref, w_ref, o_ref):
    z = (p0_ref[...] + p1_ref[...] + hs_ref[...]) * dis_ref[...] + b_ref[...]
    r = jnp.maximum(z, 0.0)
    o_ref[...] = (jnp.dot(r, w_ref[...],
                          preferred_element_type=jnp.float32) * dis_ref[...])


def _final_body(q0_ref, q1_ref, hs_ref, dis_ref, b_ref, o_ref):
    o_ref[...] = ((q0_ref[...] + q1_ref[...] + hs_ref[...]) * dis_ref[...]
                  + b_ref[...])


_ROWS = pl.BlockSpec((_BLK, D), lambda i: (i, 0))
_HSROWS = pl.BlockSpec((_BLK, D), lambda i: (i, 0))
_DISB = pl.BlockSpec((_BLK, 1), lambda i: (i, 0))
_WMAT = pl.BlockSpec((D, D), lambda i: (0, 0))
_BIAS = pl.BlockSpec((1, D), lambda i: (0, 0))


def _tc_mm_scale(x, W, dis2d):
    return pl.pallas_call(
        _mm_scale_body,
        grid=(NP // _BLK,),
        in_specs=[_ROWS, _WMAT, _DISB],
        out_specs=_ROWS,
        out_shape=jax.ShapeDtypeStruct((NP, D), jnp.float32),
    )(x, W, dis2d)


def _tc_mid(p0, p1, hs, dis2d, b, W):
    return pl.pallas_call(
        _mid_body,
        grid=(NP // _BLK,),
        in_specs=[_ROWS, _ROWS, _ROWS, _DISB, _BIAS, _WMAT],
        out_specs=_ROWS,
        out_shape=jax.ShapeDtypeStruct((NP, D), jnp.float32),
    )(p0, p1, hs, dis2d, b, W)


def _tc_final(q0, q1, hs, dis2d, b):
    return pl.pallas_call(
        _final_body,
        grid=(NP // _BLK,),
        in_specs=[_ROWS, _ROWS, _ROWS, _DISB, _BIAS],
        out_specs=_ROWS,
        out_shape=jax.ShapeDtypeStruct((NP, D), jnp.float32),
    )(q0, q1, hs, dis2d, b)


def kernel(x, edge_index, edge_weight, W1, b1, W2, b2):
    row = edge_index[0].astype(jnp.int32)
    col = edge_index[1].astype(jnp.int32)
    ew = edge_weight.astype(jnp.float32)
    pad = EP - E
    rc = jnp.bitwise_or(row, lax.shift_left(col, 14))
    rcp = jnp.concatenate([rc, jnp.zeros((pad,), jnp.int32)])
    ewp = jnp.concatenate([ew, jnp.zeros((pad,), jnp.float32)])
    xp = jnp.concatenate([x.astype(jnp.float32),
                          jnp.zeros((NP - N, D), jnp.float32)], axis=0)

    dis = _deg_dis(rcp, ewp)
    dis2d = dis.reshape(NP, 1)
    hs1 = _tc_mm_scale(xp, W1.astype(jnp.float32), dis2d)
    P = _propagate(hs1, rcp, ewp)
    hs2 = _tc_mid(P[0], P[1], hs1, dis2d, b1.reshape(1, D).astype(jnp.float32),
                  W2.astype(jnp.float32))
    Q = _propagate(hs2, rcp, ewp)
    out = _tc_final(Q[0], Q[1], hs2, dis2d, b2.reshape(1, D).astype(jnp.float32))
    return out[:N]


# speedup vs baseline: 1.4316x; 1.0197x over previous
import functools

import jax
import jax.numpy as jnp
from jax import lax
from jax.experimental import pallas as pl
from jax.experimental.pallas import tpu as pltpu
from jax.experimental.pallas import tpu_sc as plsc

N = 10000
E = 320000
D = 128
NP = 10240
NC = 2
NS = 16
NW = NC * NS
CK = 128
EW_T = 10112
DCH = EW_T // CK
EP = NW * EW_T
CH0 = 112
CH1 = 46
HB0 = CH0 // 2
HB1 = CH1 // 2
OFF1 = NS * CH0 * CK
RB = NP // NS

_mesh = plsc.VectorSubcoreMesh(core_axis_name="c", subcore_axis_name="s")


def _rsqrt_newton(v):
    r = jnp.full((16,), 1.0, jnp.float32)
    for _ in range(16):
        m = v > 4.0
        v = jnp.where(m, v * 0.25, v)
        r = jnp.where(m, r * 0.5, r)
    y = 1.0 / v
    for _ in range(6):
        y = y * (1.5 - 0.5 * v * y * y)
    return y * r


@functools.partial(
    pl.kernel,
    out_type=jax.ShapeDtypeStruct((NP,), jnp.float32),
    mesh=_mesh,
    scratch_types=[
        pltpu.VMEM((EW_T,), jnp.int32),
        pltpu.VMEM((EW_T,), jnp.float32),
        pltpu.VMEM((1, CK), jnp.int32),
        pltpu.VMEM((RB,), jnp.float32),
        pltpu.VMEM_SHARED((NP,), jnp.float32),
    ],
)
def _deg_dis(rcp, ewp, dis_out, rc_v, ew_v, colrow, dbuf, sdegf):
    cid = lax.axis_index("c")
    sid = lax.axis_index("s")

    for t in range(RB // 16):
        dbuf[pl.ds(t * 16, 16)] = jnp.zeros((16,), jnp.float32)
    pltpu.sync_copy(dbuf, sdegf.at[pl.ds(sid * RB, RB)])
    plsc.subcore_barrier()

    def do_shard(w):
        pltpu.sync_copy(rcp.at[pl.ds(w * EW_T, EW_T)], rc_v)
        pltpu.sync_copy(ewp.at[pl.ds(w * EW_T, EW_T)], ew_v)

        def chunk(j, _):
            for g in range(CK // 16):
                p16 = rc_v[pl.ds(j * CK + g * 16, 16)]
                colrow[0, pl.ds(g * 16, 16)] = lax.shift_right_logical(p16, 14)
            pltpu.sync_copy(ew_v.at[pl.ds(j * CK, CK)],
                            sdegf.at[colrow.at[0]], add=True)
            return 0

        lax.fori_loop(0, DCH, chunk, 0)

    do_shard(sid * 2)
    do_shard(sid * 2 + 1)
    plsc.subcore_barrier()

    pltpu.sync_copy(sdegf.at[pl.ds(sid * RB, RB)], dbuf)
    for t in range(RB // 16):
        v = dbuf[pl.ds(t * 16, 16)] + 1.0
        dbuf[pl.ds(t * 16, 16)] = _rsqrt_newton(v)

    @pl.when(cid == 0)
    def _():
        pltpu.sync_copy(dbuf, dis_out.at[pl.ds(sid * RB, RB)])


@functools.partial(
    pl.kernel,
    out_type=jax.ShapeDtypeStruct((NC, NP, 128), jnp.float32),
    mesh=_mesh,
    scratch_types=[
        pltpu.VMEM((HB0 * CK,), jnp.int32),
        pltpu.VMEM((HB0 * CK + 16,), jnp.float32),
        pltpu.VMEM((2, CK), jnp.int32),
        pltpu.VMEM((2, CK), jnp.int32),
        pltpu.VMEM((2, CK, 128), jnp.float32),
        pltpu.VMEM_SHARED((NP, 128), jnp.float32),
        pltpu.SemaphoreType.DMA,
        pltpu.SemaphoreType.DMA,
    ],
)
def _propagate(hs, rcp, ewp, out,
               rc_v, ew_v, rowrow, colrow, gbuf, acc, gsem, ssem):
    cid = lax.axis_index("c")
    sid = lax.axis_index("s")
    hb = jnp.where(cid == 0, HB0, HB1)

    def zrow(i, _):
        for g in range(8):
            gbuf[0, i, pl.ds(g * 16, 16)] = jnp.zeros((16,), jnp.float32)
        return 0

    lax.fori_loop(0, CK, zrow, 0)
    for k in range(RB // CK):
        pltpu.sync_copy(gbuf.at[0], acc.at[pl.ds(sid * RB + k * CK, CK)])
    plsc.subcore_barrier()

    def fill_rowrow(j):
        b = jnp.remainder(j, 2)
        for g in range(CK // 16):
            p16 = rc_v[pl.ds(j * CK + g * 16, 16)]
            rowrow[b, pl.ds(g * 16, 16)] = jnp.bitwise_and(p16, 16383)

    def issue(j):
        b = jnp.remainder(j, 2)
        pltpu.async_copy(hs.at[rowrow.at[b]], gbuf.at[b], gsem)

    def chunk(j, _):
        b = jnp.remainder(j, 2)
        bo = 1 - b
        pltpu.make_async_copy(hs.at[rowrow.at[b]], gbuf.at[b], gsem).wait()

        @pl.when(j >= 1)
        def _():
            pltpu.make_async_copy(gbuf.at[bo], acc.at[colrow.at[bo]],
                                  ssem).wait()

        @pl.when(j + 1 < hb)
        def _():
            fill_rowrow(j + 1)
            issue(j + 1)

        def scale(e, _):
            s = ew_v[pl.ds(j * CK + e, 16)][0]
            for g in range(8):
                sl = pl.ds(g * 16, 16)
                gbuf[b, e, sl] = gbuf[b, e, sl] * s
            return 0

        lax.fori_loop(0, CK, scale, 0, unroll=2)

        for g in range(CK // 16):
            p16 = rc_v[pl.ds(j * CK + g * 16, 16)]
            colrow[b, pl.ds(g * 16, 16)] = lax.shift_right_logical(p16, 14)
        pltpu.async_copy(gbuf.at[b], acc.at[colrow.at[b]], ssem, add=True)
        return 0

    for p in range(2):
        @pl.when(cid == 0)
        def _():
            off = (sid * CH0 + p * HB0) * CK
            pltpu.sync_copy(rcp.at[pl.ds(off, HB0 * CK)],
                            rc_v.at[pl.ds(0, HB0 * CK)])
            pltpu.sync_copy(ewp.at[pl.ds(off, HB0 * CK)],
                            ew_v.at[pl.ds(0, HB0 * CK)])

        @pl.when(cid == 1)
        def _():
            off = OFF1 + (sid * CH1 + p * HB1) * CK
            pltpu.sync_copy(rcp.at[pl.ds(off, HB1 * CK)],
                            rc_v.at[pl.ds(0, HB1 * CK)])
            pltpu.sync_copy(ewp.at[pl.ds(off, HB1 * CK)],
                            ew_v.at[pl.ds(0, HB1 * CK)])

        fill_rowrow(0)
        issue(0)
        lax.fori_loop(0, hb, chunk, 0)
        bl = jnp.remainder(hb - 1, 2)
        pltpu.make_async_copy(gbuf.at[bl], acc.at[colrow.at[bl]], ssem).wait()

    plsc.subcore_barrier()

    pltpu.sync_copy(acc.at[pl.ds(sid * RB, RB)], out.at[cid, pl.ds(sid * RB, RB)])


_BLK = 1024


def _mm_scale_body(x_ref, w_ref, dis_ref, o_ref):
    o_ref[...] = (jnp.dot(x_ref[...], w_ref[...],
                          preferred_element_type=jnp.float32) * dis_ref[...])


def _mid_body(p0_ref, p1_ref, hs_ref, dis_ref, b_ref, w_ref, o_ref):
    z = (p0_ref[...] + p1_ref[...] + hs_ref[...]) * dis_ref[...] + b_ref[...]
    r = jnp.maximum(z, 0.0)
    o_ref[...] = (jnp.dot(r, w_ref[...],
                          preferred_element_type=jnp.float32) * dis_ref[...])


def _final_body(q0_ref, q1_ref, hs_ref, dis_ref, b_ref, o_ref):
    o_ref[...] = ((q0_ref[...] + q1_ref[...] + hs_ref[...]) * dis_ref[...]
                  + b_ref[...])


_ROWS = pl.BlockSpec((_BLK, D), lambda i: (i, 0))
_HSROWS = pl.BlockSpec((_BLK, D), lambda i: (i, 0))
_DISB = pl.BlockSpec((_BLK, 1), lambda i: (i, 0))
_WMAT = pl.BlockSpec((D, D), lambda i: (0, 0))
_BIAS = pl.BlockSpec((1, D), lambda i: (0, 0))


def _tc_mm_scale(x, W, dis2d):
    return pl.pallas_call(
        _mm_scale_body,
        grid=(NP // _BLK,),
        in_specs=[_ROWS, _WMAT, _DISB],
        out_specs=_ROWS,
        out_shape=jax.ShapeDtypeStruct((NP, D), jnp.float32),
    )(x, W, dis2d)


def _tc_mid(p0, p1, hs, dis2d, b, W):
    return pl.pallas_call(
        _mid_body,
        grid=(NP // _BLK,),
        in_specs=[_ROWS, _ROWS, _ROWS, _DISB, _BIAS, _WMAT],
        out_specs=_ROWS,
        out_shape=jax.ShapeDtypeStruct((NP, D), jnp.float32),
    )(p0, p1, hs, dis2d, b, W)


def _tc_final(q0, q1, hs, dis2d, b):
    return pl.pallas_call(
        _final_body,
        grid=(NP // _BLK,),
        in_specs=[_ROWS, _ROWS, _ROWS, _DISB, _BIAS],
        out_specs=_ROWS,
        out_shape=jax.ShapeDtypeStruct((NP, D), jnp.float32),
    )(q0, q1, hs, dis2d, b)


def kernel(x, edge_index, edge_weight, W1, b1, W2, b2):
    row = edge_index[0].astype(jnp.int32)
    col = edge_index[1].astype(jnp.int32)
    ew = edge_weight.astype(jnp.float32)
    pad = EP - E
    rc = jnp.bitwise_or(row, lax.shift_left(col, 14))
    rcp = jnp.concatenate([rc, jnp.zeros((pad,), jnp.int32)])
    ewp = jnp.concatenate([ew, jnp.zeros((pad,), jnp.float32)])
    xp = jnp.concatenate([x.astype(jnp.float32),
                          jnp.zeros((NP - N, D), jnp.float32)], axis=0)

    dis = _deg_dis(rcp, ewp)
    dis2d = dis.reshape(NP, 1)
    hs1 = _tc_mm_scale(xp, W1.astype(jnp.float32), dis2d)
    P = _propagate(hs1, rcp, ewp)
    hs2 = _tc_mid(P[0], P[1], hs1, dis2d, b1.reshape(1, D).astype(jnp.float32),
                  W2.astype(jnp.float32))
    Q = _propagate(hs2, rcp, ewp)
    out = _tc_final(Q[0], Q[1], hs2, dis2d, b2.reshape(1, D).astype(jnp.float32))
    return out[:N]
